# baseline (speedup 1.0000x reference)
import jax
import jax.numpy as jnp
from jax.experimental import pallas as pl
from jax.experimental.pallas import tpu as pltpu

B, T, D, E = 4, 4096, 4096, 64
ROWS = 16384
BLK = 512


def _core_body(x_ref, sigt_ref, idx_ref, probs_ref,
               sigt_vmem, sgn_ref, sem):
    pltpu.make_async_copy(sigt_ref, sigt_vmem, sem).start()
    pltpu.make_async_copy(sigt_ref, sigt_vmem, sem).wait()
    sgn_ref[...] = jnp.sign(sigt_vmem[...]).astype(jnp.bfloat16)

    def step(x_blk, idx_blk, probs_blk):
        xb = x_blk[...].astype(jnp.bfloat16)
        scores = jnp.dot(xb, sgn_ref[...],
                         preferred_element_type=jnp.float32)
        m = jnp.max(scores, axis=1, keepdims=True)
        col = jax.lax.broadcasted_iota(jnp.int32, scores.shape, 1)
        idx_blk[...] = jnp.min(jnp.where(scores == m, col, E), axis=1,
                               keepdims=True)
        e = jnp.exp(scores - m)
        probs_blk[...] = e / jnp.sum(e, axis=1, keepdims=True)

    pipeline = pltpu.emit_pipeline(
        step,
        grid=(ROWS // BLK,),
        in_specs=[pl.BlockSpec((BLK, D), lambda i: (i, 0))],
        out_specs=[
            pl.BlockSpec((BLK, 1), lambda i: (i, 0)),
            pl.BlockSpec((BLK, E), lambda i: (i, 0)),
        ],
        core_axis_name="core",
        dimension_semantics=(pltpu.PARALLEL,),
    )
    pipeline(x_ref, idx_ref, probs_ref)


def kernel(x, signatures):
    x2 = x.reshape(ROWS, D)
    sigt = signatures.T
    mesh = pltpu.create_tensorcore_mesh("core", num_cores=2)

    def body(refs):
        x_ref, sigt_ref, idx_ref, probs_ref = refs

        @pl.core_map(mesh)
        def _():
            pl.run_scoped(
                lambda sigt_vmem, sgn_ref, sem: _core_body(
                    x_ref, sigt_ref, idx_ref, probs_ref,
                    sigt_vmem, sgn_ref, sem),
                pltpu.VMEM((D, E), jnp.float32),
                pltpu.VMEM((D, E), jnp.bfloat16),
                pltpu.SemaphoreType.DMA,
            )

    _, _, idx, probs = pl.run_state(body)(
        (x2, sigt,
         jnp.zeros((ROWS, 1), jnp.int32),
         jnp.zeros((ROWS, E), jnp.float32)))

    return idx.reshape(B, T), probs.reshape(B, T, E)


# speedup vs baseline: 1.1519x; 1.1519x over previous
import jax
import jax.numpy as jnp
from jax.experimental import pallas as pl
from jax.experimental.pallas import tpu as pltpu

B, T, D, E = 4, 4096, 4096, 64
ROWS = 16384
BLK = 1024


def _router_kernel(x_ref, sigt_ref, idx_ref, probs_ref):
    sgn = jnp.sign(sigt_ref[...]).astype(jnp.bfloat16)
    xb = x_ref[...].astype(jnp.bfloat16)
    scores = jnp.dot(xb, sgn, preferred_element_type=jnp.float32)

    m = jnp.max(scores, axis=1, keepdims=True)
    col = jax.lax.broadcasted_iota(jnp.int32, scores.shape, 1)
    idx_ref[...] = jnp.min(jnp.where(scores == m, col, E), axis=1,
                           keepdims=True)

    e = jnp.exp(scores - m)
    probs_ref[...] = e / jnp.sum(e, axis=1, keepdims=True)


def kernel(x, signatures):
    x2 = x.reshape(ROWS, D)
    sigt = signatures.T

    grid = (ROWS // BLK,)
    idx, probs = pl.pallas_call(
        _router_kernel,
        grid=grid,
        in_specs=[
            pl.BlockSpec((BLK, D), lambda i: (i, 0)),
            pl.BlockSpec((D, E), lambda i: (0, 0)),
        ],
        out_specs=[
            pl.BlockSpec((BLK, 1), lambda i: (i, 0)),
            pl.BlockSpec((BLK, E), lambda i: (i, 0)),
        ],
        out_shape=[
            jax.ShapeDtypeStruct((ROWS, 1), jnp.int32),
            jax.ShapeDtypeStruct((ROWS, E), jnp.float32),
        ],
        compiler_params=pltpu.CompilerParams(
            skip_device_barrier=True,
        ),
    )(x2, sigt)

    return idx.reshape(B, T), probs.reshape(B, T, E)


# speedup vs baseline: 1.2239x; 1.0626x over previous
import jax
import jax.numpy as jnp
from jax.experimental import pallas as pl
from jax.experimental.pallas import tpu as pltpu

B, T, D, E = 4, 4096, 4096, 64
ROWS = 16384
BLK = 1024


def _router_kernel(x_ref, sig_ref, idx_ref, probs_ref):
    sgn = jnp.sign(sig_ref[...]).astype(jnp.bfloat16)
    xb = x_ref[...].astype(jnp.bfloat16)
    scores = jax.lax.dot_general(
        xb, sgn, (((1,), (1,)), ((), ())),
        preferred_element_type=jnp.float32)

    m = jnp.max(scores, axis=1, keepdims=True)
    col = jax.lax.broadcasted_iota(jnp.int32, scores.shape, 1)
    idx = jnp.min(jnp.where(scores == m, col, E), axis=1)
    idx_ref[...] = idx

    e = jnp.exp(scores - m)
    probs_ref[...] = (e / jnp.sum(e, axis=1, keepdims=True)).reshape(
        probs_ref.shape)


def kernel(x, signatures):
    x2 = x.reshape(ROWS, D)

    grid = (ROWS // BLK,)
    idx, probs = pl.pallas_call(
        _router_kernel,
        grid=grid,
        in_specs=[
            pl.BlockSpec((BLK, D), lambda i: (i, 0)),
            pl.BlockSpec((E, D), lambda i: (0, 0)),
        ],
        out_specs=[
            pl.BlockSpec((BLK,), lambda i: (i,)),
            pl.BlockSpec((1, BLK, E), lambda i: (i // (T // BLK), i % (T // BLK), 0)),
        ],
        out_shape=[
            jax.ShapeDtypeStruct((ROWS,), jnp.int32),
            jax.ShapeDtypeStruct((B, T, E), jnp.float32),
        ],
    )(x2, signatures)

    return idx.reshape(B, T), probs
